# speedup vs baseline: 2.0638x; 2.0638x over previous
import functools

import jax
import jax.numpy as jnp
from jax import lax
from jax.experimental import pallas as pl
from jax.experimental.pallas import tpu as pltpu
from jax.experimental.pallas import tpu_sc as plsc

NUM_HIDDENS = 128
MAX_LEN = 8192
BATCH = 16384

NC = 2
NS = 16
NW = NC * NS
B_PER_W = BATCH // NW
CHUNK = 128
NCHUNK = B_PER_W // CHUNK


def _make_sc_gather():
    mesh = plsc.VectorSubcoreMesh(core_axis_name="c", subcore_axis_name="s")

    @functools.partial(
        pl.kernel,
        mesh=mesh,
        out_type=jax.ShapeDtypeStruct((NW, NCHUNK, CHUNK, NUM_HIDDENS), jnp.float32),
        scratch_types=[
            pltpu.VMEM((NCHUNK, CHUNK), jnp.int32),
            pltpu.VMEM((NCHUNK, CHUNK, NUM_HIDDENS), jnp.float32),
            pltpu.SemaphoreType.DMA,
            pltpu.SemaphoreType.DMA,
        ],
    )
    def sc_gather(ts_hbm, table_hbm, out_hbm, idx_v, rows_v, gsem, ssem):
        wid = lax.axis_index("s") * NC + lax.axis_index("c")
        pltpu.sync_copy(ts_hbm.at[wid], idx_v)
        for j in range(NCHUNK):
            for i in range(CHUNK // 16):
                sl = pl.ds(i * 16, 16)
                idx_v[j, sl] = (idx_v[j, sl] - 1) & (MAX_LEN - 1)
        copies = [
            pltpu.async_copy(table_hbm.at[idx_v.at[j]], rows_v.at[j], gsem)
            for j in range(NCHUNK)
        ]
        stores = []
        for j in range(NCHUNK):
            copies[j].wait()
            stores.append(pltpu.async_copy(rows_v.at[j], out_hbm.at[wid, j], ssem))
        for s in stores:
            s.wait()

    return sc_gather


_sc_gather = _make_sc_gather()


def kernel(timestep, P):
    table = P.reshape(MAX_LEN, NUM_HIDDENS)
    ts = timestep.reshape(NW, NCHUNK, CHUNK)
    out = _sc_gather(ts, table)
    return out.reshape(1, BATCH, NUM_HIDDENS)
